# baseline (speedup 1.0000x reference)
import functools

import jax
import jax.numpy as jnp
from jax import lax
from jax.experimental import pallas as pl
from jax.experimental.pallas import tpu as pltpu
from jax.experimental.pallas import tpu_sc as plsc

N = 10000
D = 128
H = 8
C = 16
E = 320000
B = 1024

NP = 10016
EDGES = E + N
NW = 32
EP = 10496
EPAD = NW * EP
GRP = EP // 128
ETOT = EPAD + 128
G0_P2 = 58
G1_P2 = 106
RPT = NP // 16
FPT = RPT * 8

_f32 = jnp.float32
_i32 = jnp.int32

_mesh = plsc.VectorSubcoreMesh(core_axis_name="c", subcore_axis_name="s")
_SC_PARAMS = pltpu.CompilerParams(
    use_tc_tiling_on_sc=False, needs_layout_passes=False)

def _ploop(lo, hi, step=1, *, unroll=1):
    if _USE_PLOOP:
        return plsc.parallel_loop(lo, hi, step, unroll=unroll)
    def deco(body):
        lax.fori_loop(lo, hi, lambda i, c: (body(i), c)[1], None)
    return deco

_USE_PLOOP = True


def _tc0_body(x_ref, w1_ref, b1_ref, wg_ref, ss_ref, sd_ref,
              xp_ref, as_ref, ad_ref):
    h = jnp.dot(x_ref[...], w1_ref[...], preferred_element_type=_f32)
    h = h + b1_ref[...]
    xp = jnp.dot(h, wg_ref[...], preferred_element_type=_f32)
    xp_ref[0] = xp[:, :64]
    xp_ref[1] = xp[:, 64:]
    as_ref[...] = jnp.dot(xp, ss_ref[...], preferred_element_type=_f32)
    ad_ref[...] = jnp.dot(xp, sd_ref[...], preferred_element_type=_f32)


def _tc0(x_pad, W1, b1, Wg, S2s, S2d):
    R = NP // 4
    return pl.pallas_call(
        _tc0_body,
        grid=(4,),
        in_specs=[
            pl.BlockSpec((R, D), lambda i: (i, 0)),
            pl.BlockSpec((D, D), lambda i: (0, 0)),
            pl.BlockSpec((1, D), lambda i: (0, 0)),
            pl.BlockSpec((D, D), lambda i: (0, 0)),
            pl.BlockSpec((D, 16), lambda i: (0, 0)),
            pl.BlockSpec((D, 16), lambda i: (0, 0)),
        ],
        out_specs=[
            pl.BlockSpec((2, R, 64), lambda i: (0, i, 0)),
            pl.BlockSpec((R, 16), lambda i: (i, 0)),
            pl.BlockSpec((R, 16), lambda i: (i, 0)),
        ],
        out_shape=[
            jax.ShapeDtypeStruct((2, NP, 64), _f32),
            jax.ShapeDtypeStruct((NP, 16), _f32),
            jax.ShapeDtypeStruct((NP, 16), _f32),
        ],
    )(x_pad, W1, b1.reshape(1, D), Wg, S2s, S2d)


def _mk_combine(project):
    def body(*refs):
        if project:
            (num_ref, s_ref, e8_ref, bg_ref, wg_ref, ss_ref, sd_ref,
             xp_ref, as_ref, ad_ref) = refs
        else:
            num_ref, s_ref, e8_ref, bg_ref, h_ref = refs
        num = jnp.concatenate(
            [num_ref[0, 0] + num_ref[1, 0], num_ref[0, 1] + num_ref[1, 1]],
            axis=-1)
        s8 = s_ref[0, :, :8] + s_ref[1, :, :8]
        sexp = jnp.dot(s8, e8_ref[...], preferred_element_type=_f32)
        h = num / (sexp + 1e-16) + bg_ref[...]
        h = jnp.where(h > 0, h, jnp.exp(h) - 1.0)
        if project:
            xp = jnp.dot(h, wg_ref[...], preferred_element_type=_f32)
            xp_ref[0] = xp[:, :64]
            xp_ref[1] = xp[:, 64:]
            as_ref[...] = jnp.dot(xp, ss_ref[...], preferred_element_type=_f32)
            ad_ref[...] = jnp.dot(xp, sd_ref[...], preferred_element_type=_f32)
        else:
            h_ref[...] = h
    return body


def _combine_project(numer2, s2, E8, bg, Wg, S2s, S2d):
    R = NP // 4
    return pl.pallas_call(
        _mk_combine(True),
        grid=(4,),
        in_specs=[
            pl.BlockSpec((2, 2, R, 64), lambda i: (0, 0, i, 0)),
            pl.BlockSpec((2, R, 16), lambda i: (0, i, 0)),
            pl.BlockSpec((8, D), lambda i: (0, 0)),
            pl.BlockSpec((1, D), lambda i: (0, 0)),
            pl.BlockSpec((D, D), lambda i: (0, 0)),
            pl.BlockSpec((D, 16), lambda i: (0, 0)),
            pl.BlockSpec((D, 16), lambda i: (0, 0)),
        ],
        out_specs=[
            pl.BlockSpec((2, R, 64), lambda i: (0, i, 0)),
            pl.BlockSpec((R, 16), lambda i: (i, 0)),
            pl.BlockSpec((R, 16), lambda i: (i, 0)),
        ],
        out_shape=[
            jax.ShapeDtypeStruct((2, NP, 64), _f32),
            jax.ShapeDtypeStruct((NP, 16), _f32),
            jax.ShapeDtypeStruct((NP, 16), _f32),
        ],
    )(numer2, s2, E8, bg.reshape(1, D), Wg, S2s, S2d)


def _combine_final(numer2, s2, E8, bg):
    R = NP // 4
    return pl.pallas_call(
        _mk_combine(False),
        grid=(4,),
        in_specs=[
            pl.BlockSpec((2, 2, R, 64), lambda i: (0, 0, i, 0)),
            pl.BlockSpec((2, R, 16), lambda i: (0, i, 0)),
            pl.BlockSpec((8, D), lambda i: (0, 0)),
            pl.BlockSpec((1, D), lambda i: (0, 0)),
        ],
        out_specs=pl.BlockSpec((R, D), lambda i: (i, 0)),
        out_shape=jax.ShapeDtypeStruct((NP, D), _f32),
    )(numer2, s2, E8, bg.reshape(1, D))


def _tcf_body(x_ref, w_ref, b_ref, o_ref):
    o_ref[...] = (jnp.dot(x_ref[...], w_ref[...], preferred_element_type=_f32)
                  + b_ref[...])


def _tc_final(concat, W3p, b3p):
    return pl.pallas_call(
        _tcf_body,
        in_specs=[
            pl.BlockSpec((B, 2 * D), lambda: (0, 0)),
            pl.BlockSpec((2 * D, 128), lambda: (0, 0)),
            pl.BlockSpec((1, 128), lambda: (0, 0)),
        ],
        out_specs=pl.BlockSpec((B, 128), lambda: (0, 0)),
        out_shape=jax.ShapeDtypeStruct((B, 128), _f32),
    )(concat, W3p, b3p.reshape(1, 128))


@functools.partial(
    pl.kernel,
    out_type=(
        jax.ShapeDtypeStruct((ETOT, 16), _f32),
        jax.ShapeDtypeStruct((NW * NP * 8 + 128,), _f32),
    ),
    mesh=_mesh,
    compiler_params=_SC_PARAMS,
    scratch_types=[
        pltpu.VMEM((2, 128), _i32),
        pltpu.VMEM((2, 128), _i32),
        pltpu.VMEM((2, 128, 16), _f32),
        pltpu.VMEM((2, 128, 16), _f32),
        pltpu.VMEM((2, 128, 16), _f32),
        pltpu.VMEM((NP * 8,), _f32),
        pltpu.SemaphoreType.DMA,
        pltpu.SemaphoreType.DMA,
        pltpu.SemaphoreType.DMA,
    ],
)
def _sc_pass1(as_hbm, ad_hbm, src_hbm, dst_hbm,
              alpha_out, mpart_out,
              srcv, dstv, asrc, adst, albuf, mpart, sem_l, sem_g, sem_a):
    wid = lax.axis_index("c") * 16 + lax.axis_index("s")
    base = wid * EP
    lane8 = lax.iota(_i32, 16) & 7
    neg = jnp.full((16,), -1e30, _f32)

    @_ploop(0, NP * 8 // 16, unroll=4)
    def _(i):
        mpart[pl.ds(i * 16, 16)] = neg

    def fire_lin(g, b):
        r = base + g * 128
        pltpu.async_copy(src_hbm.at[pl.ds(r, 128)], srcv.at[b], sem_l)
        pltpu.async_copy(dst_hbm.at[pl.ds(r, 128)], dstv.at[b], sem_l)

    def wait_lin(b):
        pltpu.make_async_copy(src_hbm.at[pl.ds(0, 128)], srcv.at[b], sem_l).wait()
        pltpu.make_async_copy(dst_hbm.at[pl.ds(0, 128)], dstv.at[b], sem_l).wait()

    def fire_g(b):
        pltpu.async_copy(as_hbm.at[srcv.at[b]], asrc.at[b], sem_g)
        pltpu.async_copy(ad_hbm.at[dstv.at[b]], adst.at[b], sem_g)

    def wait_g(b):
        pltpu.make_async_copy(as_hbm.at[srcv.at[b]], asrc.at[b], sem_g).wait()
        pltpu.make_async_copy(ad_hbm.at[dstv.at[b]], adst.at[b], sem_g).wait()

    def fire_al(g, b):
        pltpu.async_copy(albuf.at[b], alpha_out.at[pl.ds(base + g * 128, 128), :],
                         sem_a)

    def wait_al(b):
        pltpu.make_async_copy(albuf.at[b], alpha_out.at[pl.ds(0, 128), :],
                              sem_a).wait()

    def body(g, b, first):
        o = 1 - b
        fire_lin(g + 1, o)
        wait_g(b)
        if not first:
            wait_al(b)

        @_ploop(0, 128, unroll=4)
        def _(e):
            a = asrc[b, e, :] + adst[b, e, :]
            albuf[b, e, :] = jnp.where(a > 0, a, 0.2 * a)

        fire_al(g, b)
        wait_lin(o)
        fire_g(o)

        def edge(j, _):
            hi = lax.iota(_i32, 16) >> 3
            rowi = 2 * j + hi
            rows_ = 2 * j + 1 - hi
            a2 = plsc.load_gather(albuf.at[b], [rowi, lane8])
            a2s = plsc.load_gather(albuf.at[b], [rows_, lane8])
            db2 = plsc.load_gather(dstv.at[b], [rowi])
            db2s = plsc.load_gather(dstv.at[b], [rows_])
            idx = db2 * 8 + lane8
            cur = plsc.load_gather(mpart, [idx])
            mx = jnp.maximum(cur, a2)
            mx = jnp.where(db2 == db2s, jnp.maximum(mx, a2s), mx)
            plsc.store_scatter(mpart, [idx], mx)
            return _
        lax.fori_loop(0, 64, edge, None)

    pltpu.sync_copy(src_hbm.at[pl.ds(base, 128)], srcv.at[0])
    pltpu.sync_copy(dst_hbm.at[pl.ds(base, 128)], dstv.at[0])
    fire_g(0)
    body(0, 0, True)
    body(1, 1, True)

    def pair(kk, _):
        g = 2 + 2 * kk
        body(g, 0, False)
        body(g + 1, 1, False)
        return _
    lax.fori_loop(0, (GRP - 2) // 2, pair, None)

    wait_al(0)
    wait_al(1)
    wait_g(0)
    pltpu.sync_copy(mpart, mpart_out.at[pl.ds(wid * (NP * 8), NP * 8)])


MRPT = NP // 32
MFLT = MRPT * 8
MCL = 2512

@functools.partial(
    pl.kernel,
    out_type=jax.ShapeDtypeStruct((NP, 16), _f32),
    mesh=_mesh,
    compiler_params=_SC_PARAMS,
    scratch_types=[
        pltpu.VMEM((32, MCL), _f32),
        pltpu.VMEM((MCL,), _f32),
        pltpu.VMEM((MRPT, 16), _f32),
        pltpu.SemaphoreType.DMA,
    ],
)
def _sc_merge(mpart_hbm, m_out, pbuf, mslice, mdup, sem):
    wid = lax.axis_index("c") * 16 + lax.axis_index("s")
    for p in range(32):
        pltpu.async_copy(
            mpart_hbm.at[pl.ds(p * (NP * 8) + wid * MFLT, MCL)],
            pbuf.at[p], sem)
    for p in range(32):
        pltpu.make_async_copy(
            mpart_hbm.at[pl.ds(p * (NP * 8) + wid * MFLT, MCL)],
            pbuf.at[p], sem).wait()

    def vj(j, _):
        def pp(p, acc):
            return jnp.maximum(acc, pbuf[p, pl.ds(j * 16, 16)])
        acc = lax.fori_loop(0, 32, pp, jnp.full((16,), -1e30, _f32))
        mslice[pl.ds(j * 16, 16)] = acc
        return _
    lax.fori_loop(0, MCL // 16, vj, None)

    dup = lax.iota(_i32, 16) & 7

    @_ploop(0, MRPT, unroll=4)
    def _(r):
        mdup[r, :] = plsc.load_gather(mslice, [r * 8 + dup])
    pltpu.sync_copy(mdup, m_out.at[pl.ds(wid * MRPT, MRPT), :])


_CHUNKS = (1280, 1280, 1280, 1168)

@functools.partial(
    pl.kernel,
    out_type=(
        jax.ShapeDtypeStruct((2, 2, NP, 64), _f32),
        jax.ShapeDtypeStruct((2, NP, 16), _f32),
    ),
    mesh=_mesh,
    compiler_params=_SC_PARAMS,
    scratch_types=[
        pltpu.VMEM((2, 128), _i32),
        pltpu.VMEM((2, 128), _i32),
        pltpu.VMEM((2, 128), _i32),
        pltpu.VMEM((2, 128, 16), _f32),
        pltpu.VMEM((2, 128, 16), _f32),
        pltpu.VMEM((2, 128, 16), _f32),
        pltpu.VMEM((2, 128, 64), _f32),
        pltpu.VMEM_SHARED((NP, 16), _f32),
        pltpu.VMEM_SHARED((NP, 64), _f32),
        pltpu.SemaphoreType.DMA,
        pltpu.SemaphoreType.DMA,
        pltpu.SemaphoreType.DMA,
        pltpu.SemaphoreType.DMA,
        pltpu.SemaphoreType.DMA,
    ],
)
def _sc_pass2(alpha_hbm, m_hbm, src_hbm, dst_hbm, xpa_hbm, xpb_hbm,
              numer_out, s_out,
              srcv, dstv, dstw, albuf, mbuf, wbuf, rows,
              s_sp, num_sp, sem_l, sem_m, sem_x, sem_s, sem_n):
    cid = lax.axis_index("c")
    sid = lax.axis_index("s")
    zero = jnp.zeros((16,), _f32)
    r0 = sid * RPT

    def zero_bufs():
        @_ploop(0, 128 * 4, unroll=4)
        def _(i):
            rows[0, i // 4, pl.ds((i % 4) * 16, 16)] = zero

        @_ploop(0, 128, unroll=4)
        def _(i):
            wbuf[0, i, :] = zero

    def zero_accum(do_s):
        for k in range(4):
            pltpu.sync_copy(rows.at[0], num_sp.at[pl.ds(r0 + k * 128, 128), :])
        pltpu.sync_copy(rows.at[0, pl.ds(0, RPT - 512), :],
                        num_sp.at[pl.ds(r0 + 512, RPT - 512), :])
        if do_s:
            for k in range(4):
                pltpu.sync_copy(wbuf.at[0],
                                s_sp.at[pl.ds(r0 + k * 128, 128), :])
            pltpu.sync_copy(wbuf.at[0, pl.ds(0, RPT - 512), :],
                            s_sp.at[pl.ds(r0 + 512, RPT - 512), :])

    zero_bufs()
    zero_accum(True)
    plsc.subcore_barrier()

    def run_sweeps(Gc, base):
      for half, xp_hbm in ((0, xpa_hbm), (1, xpb_hbm)):
        def fire_lin(g, b):
            r = base + g * 128
            pltpu.async_copy(src_hbm.at[pl.ds(r, 128)], srcv.at[b], sem_l)
            pltpu.async_copy(dst_hbm.at[pl.ds(r, 128)], dstv.at[b], sem_l)
            pltpu.async_copy(alpha_hbm.at[pl.ds(r, 128), :], albuf.at[b], sem_l)

        def wait_lin(b):
            pltpu.make_async_copy(src_hbm.at[pl.ds(0, 128)], srcv.at[b],
                                  sem_l).wait()
            pltpu.make_async_copy(dst_hbm.at[pl.ds(0, 128)], dstv.at[b],
                                  sem_l).wait()
            pltpu.make_async_copy(alpha_hbm.at[pl.ds(0, 128), :], albuf.at[b],
                                  sem_l).wait()

        def fire_m(b):
            pltpu.async_copy(m_hbm.at[dstv.at[b]], mbuf.at[b], sem_m)

        def wait_m(b):
            pltpu.make_async_copy(m_hbm.at[dstv.at[b]], mbuf.at[b], sem_m).wait()

        def fire_x(b):
            pltpu.async_copy(xp_hbm.at[srcv.at[b]], rows.at[b], sem_x)

        def wait_x(b):
            pltpu.make_async_copy(xp_hbm.at[srcv.at[b]], rows.at[b],
                                  sem_x).wait()

        def fire_s(b):
            pltpu.async_copy(wbuf.at[b], s_sp.at[dstw.at[b]], sem_s, add=True)

        def wait_s(b):
            pltpu.make_async_copy(wbuf.at[b], s_sp.at[dstw.at[b]],
                                  sem_s).wait()

        def fire_n(b):
            pltpu.async_copy(rows.at[b], num_sp.at[dstw.at[b]], sem_n,
                             add=True)

        def wait_n(b):
            pltpu.make_async_copy(rows.at[b], num_sp.at[dstw.at[b]],
                                  sem_n).wait()

        def body(g, b, skip_s, skip_n):
            o = 1 - b
            fire_lin(g + 1, o)
            if (half == 0) and not skip_s:
                wait_s(b)
            wait_m(b)

            @_ploop(0, 128, unroll=4)
            def _(e):
                wbuf[b, e, :] = jnp.exp(albuf[b, e, :] - mbuf[b, e, :])

            @_ploop(0, 8, unroll=2)
            def _(i):
                dstw[b, pl.ds(i * 16, 16)] = dstv[b, pl.ds(i * 16, 16)]

            if half == 0:
                fire_s(b)
            wait_lin(o)
            fire_m(o)
            if not skip_n:
                wait_n(o)
            fire_x(o)
            wait_x(b)

            @_ploop(0, 128, unroll=2)
            def _(e):
                ev = jnp.full((16,), 0, _i32) + e
                for hh in range(4):
                    h = half * 4 + hh
                    sc = plsc.load_gather(
                        wbuf.at[b], [ev, jnp.full((16,), h, _i32)])
                    rows[b, e, pl.ds(hh * 16, 16)] = (
                        rows[b, e, pl.ds(hh * 16, 16)] * sc)

            fire_n(b)

        pltpu.sync_copy(src_hbm.at[pl.ds(base, 128)], srcv.at[0])
        pltpu.sync_copy(dst_hbm.at[pl.ds(base, 128)], dstv.at[0])
        pltpu.sync_copy(alpha_hbm.at[pl.ds(base, 128), :], albuf.at[0])
        fire_m(0)
        fire_x(0)
        body(0, 0, True, True)
        body(1, 1, True, False)

        def pair(kk, _):
            g = 2 + 2 * kk
            body(g, 0, False, False)
            body(g + 1, 1, False, False)
            return _
        lax.fori_loop(0, (Gc - 2) // 2, pair, None)

        if half == 0:
            wait_s(0)
            wait_s(1)
        wait_n(1)
        wait_m(0)
        wait_x(0)

        plsc.subcore_barrier()
        pltpu.sync_copy(num_sp.at[pl.ds(r0, RPT), :],
                        numer_out.at[cid, half, pl.ds(r0, RPT), :])
        if half == 0:
            pltpu.sync_copy(s_sp.at[pl.ds(r0, RPT), :],
                            s_out.at[cid, pl.ds(r0, RPT), :])
            zero_bufs()
            zero_accum(False)
            plsc.subcore_barrier()

    if G0_P2 == G1_P2:
        run_sweeps(G0_P2, (cid * 16 + sid) * (G0_P2 * 128))
    else:
        @pl.when(cid == 0)
        def _():
            run_sweeps(G0_P2, sid * (G0_P2 * 128))

        @pl.when(cid == 1)
        def _():
            run_sweeps(G1_P2, 16 * (G0_P2 * 128) + sid * (G1_P2 * 128))


@functools.partial(
    pl.kernel,
    out_type=jax.ShapeDtypeStruct((2 * B, D), _f32),
    mesh=_mesh,
    compiler_params=_SC_PARAMS,
    scratch_types=[
        pltpu.VMEM((64,), _i32),
        pltpu.VMEM((64, D), _f32),
        pltpu.SemaphoreType.DMA,
    ],
)
def _sc_gather_targets(h_hbm, tidx_hbm, out, idxv, rbuf, sem):
    wid = lax.axis_index("c") * 16 + lax.axis_index("s")
    base = wid * 64
    pltpu.sync_copy(tidx_hbm.at[pl.ds(base, 64)], idxv)
    pltpu.async_copy(h_hbm.at[idxv], rbuf, sem).wait()
    pltpu.sync_copy(rbuf, out.at[pl.ds(base, 64), :])


def _att_table(att):
    s = att[0][:, :, None] * jnp.eye(H, dtype=_f32)[:, None, :]
    s = s.reshape(H * C, H)
    return jnp.concatenate([s, s], axis=1)


def kernel(x, edge_index, target_mask, W1, b1, Wg0, att_src0, att_dst0, bg0,
           Wg1, att_src1, att_dst1, bg1, W3, b3):
    x_pad = jnp.pad(x, ((0, NP - N), (0, 0)))
    loop = jnp.arange(N, dtype=_i32)
    padn = jnp.full((ETOT - EDGES,), N, _i32)
    srcp = jnp.concatenate([edge_index[0], loop, padn])
    dstp = jnp.concatenate([edge_index[1], loop, padn])

    S2s0, S2d0 = _att_table(att_src0), _att_table(att_dst0)
    S2s1, S2d1 = _att_table(att_src1), _att_table(att_dst1)
    E8 = jnp.repeat(jnp.eye(8, dtype=_f32), 16, axis=1)
    W3p = jnp.pad(W3, ((0, 0), (0, 128 - W3.shape[1])))
    b3p = jnp.pad(b3, (0, 128 - b3.shape[0]))

    xp0, As0, Ad0 = _tc0(x_pad, W1, b1, Wg0, S2s0, S2d0)
    al0, mp0 = _sc_pass1(As0, Ad0, srcp, dstp)
    m0 = _sc_merge(mp0)
    num0, s0 = _sc_pass2(al0, m0, srcp, dstp, xp0[0], xp0[1])
    xp1, As1, Ad1 = _combine_project(num0, s0, E8, bg0, Wg1, S2s1, S2d1)
    al1, mp1 = _sc_pass1(As1, Ad1, srcp, dstp)
    m1 = _sc_merge(mp1)
    num1, s1 = _sc_pass2(al1, m1, srcp, dstp, xp1[0], xp1[1])
    hf = _combine_final(num1, s1, E8, bg1)

    tidx = target_mask.reshape(-1).astype(_i32)
    two = _sc_gather_targets(hf, tidx)
    concat = two.reshape(B, 2 * D)
    logits = _tc_final(concat, W3p, b3p)
    return logits[:, :2]


# speedup vs baseline: 1.1711x; 1.1711x over previous
import functools

import jax
import jax.numpy as jnp
from jax import lax
from jax.experimental import pallas as pl
from jax.experimental.pallas import tpu as pltpu
from jax.experimental.pallas import tpu_sc as plsc

N = 10000
D = 128
H = 8
C = 16
E = 320000
B = 1024

NP = 10016
EDGES = E + N
NW = 32
EP = 10496
EPAD = NW * EP
GRP = EP // 128
ETOT = EPAD + 128
G0_P2 = 106
G1_P2 = 58
RPT = NP // 16
FPT = RPT * 8

_f32 = jnp.float32
_i32 = jnp.int32

_mesh = plsc.VectorSubcoreMesh(core_axis_name="c", subcore_axis_name="s")
_SC_PARAMS = pltpu.CompilerParams(
    use_tc_tiling_on_sc=False, needs_layout_passes=False)

def _ploop(lo, hi, step=1, *, unroll=1):
    if _USE_PLOOP:
        return plsc.parallel_loop(lo, hi, step, unroll=unroll)
    def deco(body):
        lax.fori_loop(lo, hi, lambda i, c: (body(i), c)[1], None)
    return deco

_USE_PLOOP = True


def _tc0_body(x_ref, w1_ref, b1_ref, wg_ref, ss_ref, sd_ref,
              xp_ref, as_ref, ad_ref):
    h = jnp.dot(x_ref[...], w1_ref[...], preferred_element_type=_f32)
    h = h + b1_ref[...]
    xp = jnp.dot(h, wg_ref[...], preferred_element_type=_f32)
    xp_ref[0] = xp[:, :64]
    xp_ref[1] = xp[:, 64:]
    as_ref[...] = jnp.dot(xp, ss_ref[...], preferred_element_type=_f32)
    ad_ref[...] = jnp.dot(xp, sd_ref[...], preferred_element_type=_f32)


def _tc0(x_pad, W1, b1, Wg, S2s, S2d):
    R = NP // 4
    return pl.pallas_call(
        _tc0_body,
        grid=(4,),
        in_specs=[
            pl.BlockSpec((R, D), lambda i: (i, 0)),
            pl.BlockSpec((D, D), lambda i: (0, 0)),
            pl.BlockSpec((1, D), lambda i: (0, 0)),
            pl.BlockSpec((D, D), lambda i: (0, 0)),
            pl.BlockSpec((D, 16), lambda i: (0, 0)),
            pl.BlockSpec((D, 16), lambda i: (0, 0)),
        ],
        out_specs=[
            pl.BlockSpec((2, R, 64), lambda i: (0, i, 0)),
            pl.BlockSpec((R, 16), lambda i: (i, 0)),
            pl.BlockSpec((R, 16), lambda i: (i, 0)),
        ],
        out_shape=[
            jax.ShapeDtypeStruct((2, NP, 64), _f32),
            jax.ShapeDtypeStruct((NP, 16), _f32),
            jax.ShapeDtypeStruct((NP, 16), _f32),
        ],
    )(x_pad, W1, b1.reshape(1, D), Wg, S2s, S2d)


def _mk_combine(project):
    def body(*refs):
        if project:
            (num_ref, s_ref, e8_ref, bg_ref, wg_ref, ss_ref, sd_ref,
             xp_ref, as_ref, ad_ref) = refs
        else:
            num_ref, s_ref, e8_ref, bg_ref, h_ref = refs
        num = jnp.concatenate(
            [num_ref[0, 0] + num_ref[1, 0], num_ref[0, 1] + num_ref[1, 1]],
            axis=-1)
        s8 = s_ref[0, :, :8] + s_ref[1, :, :8]
        sexp = jnp.dot(s8, e8_ref[...], preferred_element_type=_f32)
        h = num / (sexp + 1e-16) + bg_ref[...]
        h = jnp.where(h > 0, h, jnp.exp(h) - 1.0)
        if project:
            xp = jnp.dot(h, wg_ref[...], preferred_element_type=_f32)
            xp_ref[0] = xp[:, :64]
            xp_ref[1] = xp[:, 64:]
            as_ref[...] = jnp.dot(xp, ss_ref[...], preferred_element_type=_f32)
            ad_ref[...] = jnp.dot(xp, sd_ref[...], preferred_element_type=_f32)
        else:
            h_ref[...] = h
    return body


def _combine_project(numer2, s2, E8, bg, Wg, S2s, S2d):
    R = NP // 4
    return pl.pallas_call(
        _mk_combine(True),
        grid=(4,),
        in_specs=[
            pl.BlockSpec((2, 2, R, 64), lambda i: (0, 0, i, 0)),
            pl.BlockSpec((2, R, 16), lambda i: (0, i, 0)),
            pl.BlockSpec((8, D), lambda i: (0, 0)),
            pl.BlockSpec((1, D), lambda i: (0, 0)),
            pl.BlockSpec((D, D), lambda i: (0, 0)),
            pl.BlockSpec((D, 16), lambda i: (0, 0)),
            pl.BlockSpec((D, 16), lambda i: (0, 0)),
        ],
        out_specs=[
            pl.BlockSpec((2, R, 64), lambda i: (0, i, 0)),
            pl.BlockSpec((R, 16), lambda i: (i, 0)),
            pl.BlockSpec((R, 16), lambda i: (i, 0)),
        ],
        out_shape=[
            jax.ShapeDtypeStruct((2, NP, 64), _f32),
            jax.ShapeDtypeStruct((NP, 16), _f32),
            jax.ShapeDtypeStruct((NP, 16), _f32),
        ],
    )(numer2, s2, E8, bg.reshape(1, D), Wg, S2s, S2d)


def _combine_final(numer2, s2, E8, bg):
    R = NP // 4
    return pl.pallas_call(
        _mk_combine(False),
        grid=(4,),
        in_specs=[
            pl.BlockSpec((2, 2, R, 64), lambda i: (0, 0, i, 0)),
            pl.BlockSpec((2, R, 16), lambda i: (0, i, 0)),
            pl.BlockSpec((8, D), lambda i: (0, 0)),
            pl.BlockSpec((1, D), lambda i: (0, 0)),
        ],
        out_specs=pl.BlockSpec((R, D), lambda i: (i, 0)),
        out_shape=jax.ShapeDtypeStruct((NP, D), _f32),
    )(numer2, s2, E8, bg.reshape(1, D))


def _tcf_body(x_ref, w_ref, b_ref, o_ref):
    o_ref[...] = (jnp.dot(x_ref[...], w_ref[...], preferred_element_type=_f32)
                  + b_ref[...])


def _tc_final(concat, W3p, b3p):
    return pl.pallas_call(
        _tcf_body,
        in_specs=[
            pl.BlockSpec((B, 2 * D), lambda: (0, 0)),
            pl.BlockSpec((2 * D, 128), lambda: (0, 0)),
            pl.BlockSpec((1, 128), lambda: (0, 0)),
        ],
        out_specs=pl.BlockSpec((B, 128), lambda: (0, 0)),
        out_shape=jax.ShapeDtypeStruct((B, 128), _f32),
    )(concat, W3p, b3p.reshape(1, 128))


@functools.partial(
    pl.kernel,
    out_type=(
        jax.ShapeDtypeStruct((ETOT, 16), _f32),
        jax.ShapeDtypeStruct((NW * NP * 8 + 128,), _f32),
    ),
    mesh=_mesh,
    compiler_params=_SC_PARAMS,
    scratch_types=[
        pltpu.VMEM((2, 128), _i32),
        pltpu.VMEM((2, 128), _i32),
        pltpu.VMEM((2, 128, 16), _f32),
        pltpu.VMEM((2, 128, 16), _f32),
        pltpu.VMEM((2, 128, 16), _f32),
        pltpu.VMEM((NP * 8,), _f32),
        pltpu.SemaphoreType.DMA,
        pltpu.SemaphoreType.DMA,
        pltpu.SemaphoreType.DMA,
    ],
)
def _sc_pass1(as_hbm, ad_hbm, src_hbm, dst_hbm,
              alpha_out, mpart_out,
              srcv, dstv, asrc, adst, albuf, mpart, sem_l, sem_g, sem_a):
    wid = lax.axis_index("c") * 16 + lax.axis_index("s")
    base = wid * EP
    lane8 = lax.iota(_i32, 16) & 7
    neg = jnp.full((16,), -1e30, _f32)

    @_ploop(0, NP * 8 // 16, unroll=4)
    def _(i):
        mpart[pl.ds(i * 16, 16)] = neg

    def fire_lin(g, b):
        r = base + g * 128
        pltpu.async_copy(src_hbm.at[pl.ds(r, 128)], srcv.at[b], sem_l)
        pltpu.async_copy(dst_hbm.at[pl.ds(r, 128)], dstv.at[b], sem_l)

    def wait_lin(b):
        pltpu.make_async_copy(src_hbm.at[pl.ds(0, 128)], srcv.at[b], sem_l).wait()
        pltpu.make_async_copy(dst_hbm.at[pl.ds(0, 128)], dstv.at[b], sem_l).wait()

    def fire_g(b):
        pltpu.async_copy(as_hbm.at[srcv.at[b]], asrc.at[b], sem_g)
        pltpu.async_copy(ad_hbm.at[dstv.at[b]], adst.at[b], sem_g)

    def wait_g(b):
        pltpu.make_async_copy(as_hbm.at[srcv.at[b]], asrc.at[b], sem_g).wait()
        pltpu.make_async_copy(ad_hbm.at[dstv.at[b]], adst.at[b], sem_g).wait()

    def fire_al(g, b):
        pltpu.async_copy(albuf.at[b], alpha_out.at[pl.ds(base + g * 128, 128), :],
                         sem_a)

    def wait_al(b):
        pltpu.make_async_copy(albuf.at[b], alpha_out.at[pl.ds(0, 128), :],
                              sem_a).wait()

    def body(g, b, first):
        o = 1 - b
        fire_lin(g + 1, o)
        wait_g(b)
        if not first:
            wait_al(b)

        @_ploop(0, 128, unroll=4)
        def _(e):
            a = asrc[b, e, :] + adst[b, e, :]
            albuf[b, e, :] = jnp.where(a > 0, a, 0.2 * a)

        fire_al(g, b)
        wait_lin(o)
        fire_g(o)

        def edge(j, _):
            hi = lax.iota(_i32, 16) >> 3
            rowi = 2 * j + hi
            rows_ = 2 * j + 1 - hi
            a2 = plsc.load_gather(albuf.at[b], [rowi, lane8])
            a2s = plsc.load_gather(albuf.at[b], [rows_, lane8])
            db2 = plsc.load_gather(dstv.at[b], [rowi])
            db2s = plsc.load_gather(dstv.at[b], [rows_])
            idx = db2 * 8 + lane8
            cur = plsc.load_gather(mpart, [idx])
            mx = jnp.maximum(cur, a2)
            mx = jnp.where(db2 == db2s, jnp.maximum(mx, a2s), mx)
            plsc.store_scatter(mpart, [idx], mx)
            return _
        lax.fori_loop(0, 64, edge, None)

    pltpu.sync_copy(src_hbm.at[pl.ds(base, 128)], srcv.at[0])
    pltpu.sync_copy(dst_hbm.at[pl.ds(base, 128)], dstv.at[0])
    fire_g(0)
    body(0, 0, True)
    body(1, 1, True)

    def pair(kk, _):
        g = 2 + 2 * kk
        body(g, 0, False)
        body(g + 1, 1, False)
        return _
    lax.fori_loop(0, (GRP - 2) // 2, pair, None)

    wait_al(0)
    wait_al(1)
    wait_g(0)
    pltpu.sync_copy(mpart, mpart_out.at[pl.ds(wid * (NP * 8), NP * 8)])


MRPT = NP // 32
MFLT = MRPT * 8
MCL = 2512

@functools.partial(
    pl.kernel,
    out_type=jax.ShapeDtypeStruct((NP, 16), _f32),
    mesh=_mesh,
    compiler_params=_SC_PARAMS,
    scratch_types=[
        pltpu.VMEM((32, MCL), _f32),
        pltpu.VMEM((MCL,), _f32),
        pltpu.VMEM((MRPT, 16), _f32),
        pltpu.SemaphoreType.DMA,
    ],
)
def _sc_merge(mpart_hbm, m_out, pbuf, mslice, mdup, sem):
    wid = lax.axis_index("c") * 16 + lax.axis_index("s")
    for p in range(32):
        pltpu.async_copy(
            mpart_hbm.at[pl.ds(p * (NP * 8) + wid * MFLT, MCL)],
            pbuf.at[p], sem)
    for p in range(32):
        pltpu.make_async_copy(
            mpart_hbm.at[pl.ds(p * (NP * 8) + wid * MFLT, MCL)],
            pbuf.at[p], sem).wait()

    def vj(j, _):
        def pp(p, acc):
            return jnp.maximum(acc, pbuf[p, pl.ds(j * 16, 16)])
        acc = lax.fori_loop(0, 32, pp, jnp.full((16,), -1e30, _f32))
        mslice[pl.ds(j * 16, 16)] = acc
        return _
    lax.fori_loop(0, MCL // 16, vj, None)

    dup = lax.iota(_i32, 16) & 7

    @_ploop(0, MRPT, unroll=4)
    def _(r):
        mdup[r, :] = plsc.load_gather(mslice, [r * 8 + dup])
    pltpu.sync_copy(mdup, m_out.at[pl.ds(wid * MRPT, MRPT), :])


_CHUNKS = (1280, 1280, 1280, 1168)

@functools.partial(
    pl.kernel,
    out_type=(
        jax.ShapeDtypeStruct((2, 2, NP, 64), _f32),
        jax.ShapeDtypeStruct((2, NP, 16), _f32),
    ),
    mesh=_mesh,
    compiler_params=_SC_PARAMS,
    scratch_types=[
        pltpu.VMEM((2, 128), _i32),
        pltpu.VMEM((2, 128), _i32),
        pltpu.VMEM((2, 128), _i32),
        pltpu.VMEM((2, 128, 16), _f32),
        pltpu.VMEM((2, 128, 16), _f32),
        pltpu.VMEM((2, 128, 16), _f32),
        pltpu.VMEM((2, 128, 64), _f32),
        pltpu.VMEM_SHARED((NP, 16), _f32),
        pltpu.VMEM_SHARED((NP, 64), _f32),
        pltpu.SemaphoreType.DMA,
        pltpu.SemaphoreType.DMA,
        pltpu.SemaphoreType.DMA,
        pltpu.SemaphoreType.DMA,
        pltpu.SemaphoreType.DMA,
    ],
)
def _sc_pass2(alpha_hbm, m_hbm, src_hbm, dst_hbm, xpa_hbm, xpb_hbm,
              numer_out, s_out,
              srcv, dstv, dstw, albuf, mbuf, wbuf, rows,
              s_sp, num_sp, sem_l, sem_m, sem_x, sem_s, sem_n):
    cid = lax.axis_index("c")
    sid = lax.axis_index("s")
    zero = jnp.zeros((16,), _f32)
    r0 = sid * RPT

    def zero_bufs():
        @_ploop(0, 128 * 4, unroll=4)
        def _(i):
            rows[0, i // 4, pl.ds((i % 4) * 16, 16)] = zero

        @_ploop(0, 128, unroll=4)
        def _(i):
            wbuf[0, i, :] = zero

    def zero_accum(do_s):
        for k in range(4):
            pltpu.sync_copy(rows.at[0], num_sp.at[pl.ds(r0 + k * 128, 128), :])
        pltpu.sync_copy(rows.at[0, pl.ds(0, RPT - 512), :],
                        num_sp.at[pl.ds(r0 + 512, RPT - 512), :])
        if do_s:
            for k in range(4):
                pltpu.sync_copy(wbuf.at[0],
                                s_sp.at[pl.ds(r0 + k * 128, 128), :])
            pltpu.sync_copy(wbuf.at[0, pl.ds(0, RPT - 512), :],
                            s_sp.at[pl.ds(r0 + 512, RPT - 512), :])

    zero_bufs()
    zero_accum(True)
    plsc.subcore_barrier()

    def run_sweeps(Gc, base):
      for half, xp_hbm in ((0, xpa_hbm), (1, xpb_hbm)):
        def fire_lin(g, b):
            r = base + g * 128
            pltpu.async_copy(src_hbm.at[pl.ds(r, 128)], srcv.at[b], sem_l)
            pltpu.async_copy(dst_hbm.at[pl.ds(r, 128)], dstv.at[b], sem_l)
            pltpu.async_copy(alpha_hbm.at[pl.ds(r, 128), :], albuf.at[b], sem_l)

        def wait_lin(b):
            pltpu.make_async_copy(src_hbm.at[pl.ds(0, 128)], srcv.at[b],
                                  sem_l).wait()
            pltpu.make_async_copy(dst_hbm.at[pl.ds(0, 128)], dstv.at[b],
                                  sem_l).wait()
            pltpu.make_async_copy(alpha_hbm.at[pl.ds(0, 128), :], albuf.at[b],
                                  sem_l).wait()

        def fire_m(b):
            pltpu.async_copy(m_hbm.at[dstv.at[b]], mbuf.at[b], sem_m)

        def wait_m(b):
            pltpu.make_async_copy(m_hbm.at[dstv.at[b]], mbuf.at[b], sem_m).wait()

        def fire_x(b):
            pltpu.async_copy(xp_hbm.at[srcv.at[b]], rows.at[b], sem_x)

        def wait_x(b):
            pltpu.make_async_copy(xp_hbm.at[srcv.at[b]], rows.at[b],
                                  sem_x).wait()

        def fire_s(b):
            pltpu.async_copy(wbuf.at[b], s_sp.at[dstw.at[b]], sem_s, add=True)

        def wait_s(b):
            pltpu.make_async_copy(wbuf.at[b], s_sp.at[dstw.at[b]],
                                  sem_s).wait()

        def fire_n(b):
            pltpu.async_copy(rows.at[b], num_sp.at[dstw.at[b]], sem_n,
                             add=True)

        def wait_n(b):
            pltpu.make_async_copy(rows.at[b], num_sp.at[dstw.at[b]],
                                  sem_n).wait()

        def body(g, b, skip_s, skip_n):
            o = 1 - b
            fire_lin(g + 1, o)
            if (half == 0) and not skip_s:
                wait_s(b)
            wait_m(b)

            @_ploop(0, 128, unroll=4)
            def _(e):
                wbuf[b, e, :] = jnp.exp(albuf[b, e, :] - mbuf[b, e, :])

            @_ploop(0, 8, unroll=2)
            def _(i):
                dstw[b, pl.ds(i * 16, 16)] = dstv[b, pl.ds(i * 16, 16)]

            if half == 0:
                fire_s(b)
            wait_lin(o)
            fire_m(o)
            if not skip_n:
                wait_n(o)
            fire_x(o)
            wait_x(b)

            @_ploop(0, 128, unroll=2)
            def _(e):
                ev = jnp.full((16,), 0, _i32) + e
                for hh in range(4):
                    h = half * 4 + hh
                    sc = plsc.load_gather(
                        wbuf.at[b], [ev, jnp.full((16,), h, _i32)])
                    rows[b, e, pl.ds(hh * 16, 16)] = (
                        rows[b, e, pl.ds(hh * 16, 16)] * sc)

            fire_n(b)

        pltpu.sync_copy(src_hbm.at[pl.ds(base, 128)], srcv.at[0])
        pltpu.sync_copy(dst_hbm.at[pl.ds(base, 128)], dstv.at[0])
        pltpu.sync_copy(alpha_hbm.at[pl.ds(base, 128), :], albuf.at[0])
        fire_m(0)
        fire_x(0)
        body(0, 0, True, True)
        body(1, 1, True, False)

        def pair(kk, _):
            g = 2 + 2 * kk
            body(g, 0, False, False)
            body(g + 1, 1, False, False)
            return _
        lax.fori_loop(0, (Gc - 2) // 2, pair, None)

        if half == 0:
            wait_s(0)
            wait_s(1)
        wait_n(1)
        wait_m(0)
        wait_x(0)

        plsc.subcore_barrier()
        pltpu.sync_copy(num_sp.at[pl.ds(r0, RPT), :],
                        numer_out.at[cid, half, pl.ds(r0, RPT), :])
        if half == 0:
            pltpu.sync_copy(s_sp.at[pl.ds(r0, RPT), :],
                            s_out.at[cid, pl.ds(r0, RPT), :])
            zero_bufs()
            zero_accum(False)
            plsc.subcore_barrier()

    if G0_P2 == G1_P2:
        run_sweeps(G0_P2, (cid * 16 + sid) * (G0_P2 * 128))
    else:
        @pl.when(cid == 0)
        def _():
            run_sweeps(G0_P2, sid * (G0_P2 * 128))

        @pl.when(cid == 1)
        def _():
            run_sweeps(G1_P2, 16 * (G0_P2 * 128) + sid * (G1_P2 * 128))


@functools.partial(
    pl.kernel,
    out_type=jax.ShapeDtypeStruct((2 * B, D), _f32),
    mesh=_mesh,
    compiler_params=_SC_PARAMS,
    scratch_types=[
        pltpu.VMEM((64,), _i32),
        pltpu.VMEM((64, D), _f32),
        pltpu.SemaphoreType.DMA,
    ],
)
def _sc_gather_targets(h_hbm, tidx_hbm, out, idxv, rbuf, sem):
    wid = lax.axis_index("c") * 16 + lax.axis_index("s")
    base = wid * 64
    pltpu.sync_copy(tidx_hbm.at[pl.ds(base, 64)], idxv)
    pltpu.async_copy(h_hbm.at[idxv], rbuf, sem).wait()
    pltpu.sync_copy(rbuf, out.at[pl.ds(base, 64), :])


def _att_table(att):
    s = att[0][:, :, None] * jnp.eye(H, dtype=_f32)[:, None, :]
    s = s.reshape(H * C, H)
    return jnp.concatenate([s, s], axis=1)


def kernel(x, edge_index, target_mask, W1, b1, Wg0, att_src0, att_dst0, bg0,
           Wg1, att_src1, att_dst1, bg1, W3, b3):
    x_pad = jnp.pad(x, ((0, NP - N), (0, 0)))
    loop = jnp.arange(N, dtype=_i32)
    padn = jnp.full((ETOT - EDGES,), N, _i32)
    srcp = jnp.concatenate([edge_index[0], loop, padn])
    dstp = jnp.concatenate([edge_index[1], loop, padn])

    S2s0, S2d0 = _att_table(att_src0), _att_table(att_dst0)
    S2s1, S2d1 = _att_table(att_src1), _att_table(att_dst1)
    E8 = jnp.repeat(jnp.eye(8, dtype=_f32), 16, axis=1)
    W3p = jnp.pad(W3, ((0, 0), (0, 128 - W3.shape[1])))
    b3p = jnp.pad(b3, (0, 128 - b3.shape[0]))

    xp0, As0, Ad0 = _tc0(x_pad, W1, b1, Wg0, S2s0, S2d0)
    al0, mp0 = _sc_pass1(As0, Ad0, srcp, dstp)
    m0 = _sc_merge(mp0)
    num0, s0 = _sc_pass2(al0, m0, srcp, dstp, xp0[0], xp0[1])
    xp1, As1, Ad1 = _combine_project(num0, s0, E8, bg0, Wg1, S2s1, S2d1)
    al1, mp1 = _sc_pass1(As1, Ad1, srcp, dstp)
    m1 = _sc_merge(mp1)
    num1, s1 = _sc_pass2(al1, m1, srcp, dstp, xp1[0], xp1[1])
    hf = _combine_final(num1, s1, E8, bg1)

    tidx = target_mask.reshape(-1).astype(_i32)
    two = _sc_gather_targets(hf, tidx)
    concat = two.reshape(B, 2 * D)
    logits = _tc_final(concat, W3p, b3p)
    return logits[:, :2]
